# baseline (speedup 1.0000x reference)
import functools

import jax
import jax.numpy as jnp
from jax import lax
from jax.experimental import pallas as pl
from jax.experimental.pallas import tpu as pltpu
from jax.experimental.pallas import tpu_sc as plsc

NC = 2
NS = 16
L = 16
NW = NC * NS

B = 16384
D = 32
GW = 128
BPW = B // NW
CHUNK = 128
NCHUNK = BPW // CHUNK


def _dot_kernel(uidx_hbm, iidx_hbm, user_t_hbm, item_t_hbm, out_hbm,
                uidx_v, iidx_v, ugid_v, igid_v, uoff_v, ioff_v,
                ubuf, vbuf, out_v, usem, isem):
    wid = lax.axis_index("s") * NC + lax.axis_index("c")
    base = wid * BPW

    pltpu.sync_copy(uidx_hbm.at[wid], uidx_v)
    pltpu.sync_copy(iidx_hbm.at[wid], iidx_v)

    def split_body(j, _):
        c = j // (CHUNK // L)
        s = (j % (CHUNK // L)) * L
        x = uidx_v[c, pl.ds(s, L)]
        ugid_v[c, pl.ds(s, L)] = lax.shift_right_logical(x, 2)
        uoff_v[c, pl.ds(s, L)] = lax.shift_left(x & 3, 5)
        y = iidx_v[c, pl.ds(s, L)]
        igid_v[c, pl.ds(s, L)] = lax.shift_right_logical(y, 2)
        ioff_v[c, pl.ds(s, L)] = lax.shift_left(y & 3, 5)
        return 0

    lax.fori_loop(0, NCHUNK * (CHUNK // L), split_body, 0)

    def fire(c):
        b = c % 2
        uc = pltpu.async_copy(user_t_hbm.at[ugid_v.at[c]], ubuf.at[b], usem)
        ic = pltpu.async_copy(item_t_hbm.at[igid_v.at[c]], vbuf.at[b], isem)
        return uc, ic

    def compute(c):
        b = c % 2
        bvec = jnp.full((L,), b, jnp.int32)

        def gbody(g, _):
            rows = g * L + lax.iota(jnp.int32, L)
            ou = uoff_v[c, pl.ds(g * L, L)]
            oi = ioff_v[c, pl.ds(g * L, L)]
            acc = jnp.zeros((L,), jnp.float32)
            for d in range(D):
                u = plsc.load_gather(ubuf, [bvec, rows, ou + d])
                v = plsc.load_gather(vbuf, [bvec, rows, oi + d])
                acc = acc + u * v
            out_v[pl.ds(c * CHUNK + g * L, L)] = acc
            return 0

        lax.fori_loop(0, CHUNK // L, gbody, 0)

    copies = {0: fire(0)}
    for c in range(NCHUNK):
        if c + 1 < NCHUNK:
            copies[c + 1] = fire(c + 1)
        cu, ci = copies[c]
        cu.wait()
        ci.wait()
        compute(c)

    pltpu.sync_copy(out_v, out_hbm.at[pl.ds(base, BPW)])


@jax.jit
def _run(user_idx, item_idx, user_w, item_w):
    mesh = plsc.VectorSubcoreMesh(core_axis_name="c", subcore_axis_name="s")
    k = functools.partial(
        pl.kernel,
        out_type=jax.ShapeDtypeStruct((B,), jnp.float32),
        mesh=mesh,
        compiler_params=pltpu.CompilerParams(
            needs_layout_passes=False, use_tc_tiling_on_sc=False),
        scratch_types=[
            pltpu.VMEM((NCHUNK, CHUNK), jnp.int32),
            pltpu.VMEM((NCHUNK, CHUNK), jnp.int32),
            pltpu.VMEM((NCHUNK, CHUNK), jnp.int32),
            pltpu.VMEM((NCHUNK, CHUNK), jnp.int32),
            pltpu.VMEM((NCHUNK, CHUNK), jnp.int32),
            pltpu.VMEM((NCHUNK, CHUNK), jnp.int32),
            pltpu.VMEM((2, CHUNK, GW), jnp.float32),
            pltpu.VMEM((2, CHUNK, GW), jnp.float32),
            pltpu.VMEM((BPW,), jnp.float32),
            pltpu.SemaphoreType.DMA,
            pltpu.SemaphoreType.DMA,
        ],
    )(_dot_kernel)
    uidx = user_idx.reshape(NW, NCHUNK, CHUNK)
    iidx = item_idx.reshape(NW, NCHUNK, CHUNK)
    user_t = user_w.reshape(-1, GW)
    item_t = item_w.reshape(-1, GW)
    return k(uidx, iidx, user_t, item_t)


def kernel(user_idx, item_idx, user_w, item_w):
    return _run(user_idx, item_idx, user_w, item_w)


# speedup vs baseline: 4.4207x; 4.4207x over previous
import functools

import jax
import jax.numpy as jnp
from jax import lax
from jax.experimental import pallas as pl
from jax.experimental.pallas import tpu as pltpu
from jax.experimental.pallas import tpu_sc as plsc

NC = 2
NS = 16
L = 16
NW = NC * NS

B = 16384
D = 32
BPW = B // NW
KQ = 8
FW = 128


def _dot_kernel(uidx_hbm, iidx_hbm, u_t_hbm, i_t_hbm, out_hbm,
                uidx_vm, iidx_vm, ubuf, vbuf, ustage, vstage, out_v,
                usem, isem):
    wid = lax.axis_index("s") * NC + lax.axis_index("c")
    base = wid * BPW

    pltpu.sync_copy(uidx_hbm.at[pl.ds(base, BPW)], uidx_vm)
    pltpu.sync_copy(iidx_hbm.at[pl.ds(base, BPW)], iidx_vm)

    rows0 = lax.iota(jnp.int32, L)

    def off_of(r):
        return pl.multiple_of((r >> 7) << 7, 128)

    def fire(slot, ru, ri):
        pltpu.async_copy(
            u_t_hbm.at[:, pl.ds(off_of(ru), FW)], ubuf.at[slot], usem)
        pltpu.async_copy(
            i_t_hbm.at[:, pl.ds(off_of(ri), FW)], vbuf.at[slot], isem)

    ruv0 = uidx_vm[pl.ds(0, L)]
    riv0 = iidx_vm[pl.ds(0, L)]
    for j in range(KQ):
        fire(j, ruv0[j], riv0[j])

    def gbody(g, _):
        ruv = uidx_vm[pl.ds(g * L, L)]
        riv = iidx_vm[pl.ds(g * L, L)]
        nbase = jnp.minimum((g + 1) * L, BPW - L)
        rnu = uidx_vm[pl.ds(nbase, L)]
        rni = iidx_vm[pl.ds(nbase, L)]
        for j in range(L):
            b = g * L + j
            slot = lax.rem(b, KQ)
            pltpu.make_async_copy(
                u_t_hbm.at[:, pl.ds(0, FW)], ubuf.at[slot], usem).wait()
            pltpu.make_async_copy(
                i_t_hbm.at[:, pl.ds(0, FW)], vbuf.at[slot], isem).wait()
            ru = ruv[j]
            ri = riv[j]
            slot_v = jnp.full((L,), slot, jnp.int32)
            cu = jnp.full((L,), ru & 127, jnp.int32)
            cv = jnp.full((L,), ri & 127, jnp.int32)
            jv = jnp.full((L,), j, jnp.int32)
            u0 = plsc.load_gather(ubuf, [slot_v, rows0, cu])
            u1 = plsc.load_gather(ubuf, [slot_v, rows0 + L, cu])
            v0 = plsc.load_gather(vbuf, [slot_v, rows0, cv])
            v1 = plsc.load_gather(vbuf, [slot_v, rows0 + L, cv])
            plsc.store_scatter(ustage, [rows0, jv], u0)
            plsc.store_scatter(ustage, [rows0 + L, jv], u1)
            plsc.store_scatter(vstage, [rows0, jv], v0)
            plsc.store_scatter(vstage, [rows0 + L, jv], v1)

            if j < KQ:
                nru, nri = ruv[j + KQ], riv[j + KQ]
            else:
                nru, nri = rnu[j - KQ], rni[j - KQ]

            @pl.when(b + KQ < BPW)
            def _():
                fire(slot, nru, nri)

        acc = jnp.zeros((L,), jnp.float32)
        for d in range(D):
            acc = acc + ustage[d] * vstage[d]
        out_v[pl.ds(g * L, L)] = acc
        return 0

    lax.fori_loop(0, BPW // L, gbody, 0)

    pltpu.sync_copy(out_v, out_hbm.at[pl.ds(base, BPW)])


@jax.jit
def _run(user_idx, item_idx, user_w, item_w):
    mesh = plsc.VectorSubcoreMesh(core_axis_name="c", subcore_axis_name="s")
    k = functools.partial(
        pl.kernel,
        out_type=jax.ShapeDtypeStruct((B,), jnp.float32),
        mesh=mesh,
        compiler_params=pltpu.CompilerParams(
            needs_layout_passes=False, use_tc_tiling_on_sc=True),
        scratch_types=[
            pltpu.VMEM((BPW,), jnp.int32),
            pltpu.VMEM((BPW,), jnp.int32),
            pltpu.VMEM((KQ, D, FW), jnp.float32),
            pltpu.VMEM((KQ, D, FW), jnp.float32),
            pltpu.VMEM((D, L), jnp.float32),
            pltpu.VMEM((D, L), jnp.float32),
            pltpu.VMEM((BPW,), jnp.float32),
            pltpu.SemaphoreType.DMA,
            pltpu.SemaphoreType.DMA,
        ],
    )(_dot_kernel)
    return k(user_idx, item_idx, user_w.T, item_w.T)


def kernel(user_idx, item_idx, user_w, item_w):
    return _run(user_idx, item_idx, user_w, item_w)


# speedup vs baseline: 4.4344x; 1.0031x over previous
import functools

import jax
import jax.numpy as jnp
from jax import lax
from jax.experimental import pallas as pl
from jax.experimental.pallas import tpu as pltpu
from jax.experimental.pallas import tpu_sc as plsc

NC = 2
NS = 16
L = 16
NW = NC * NS

B = 16384
D = 32
BPW = B // NW
KQ = 12
FW = 128


def _dot_kernel(uidx_hbm, iidx_hbm, u_t_hbm, i_t_hbm, out_hbm,
                uidx_vm, iidx_vm, ubuf, vbuf, ustage, vstage, out_v,
                usem, isem):
    wid = lax.axis_index("s") * NC + lax.axis_index("c")
    base = wid * BPW

    pltpu.sync_copy(uidx_hbm.at[pl.ds(base, BPW)], uidx_vm)
    pltpu.sync_copy(iidx_hbm.at[pl.ds(base, BPW)], iidx_vm)

    rows0 = lax.iota(jnp.int32, L)

    def off_of(r):
        return pl.multiple_of((r >> 7) << 7, 128)

    def fire(slot, ru, ri):
        pltpu.async_copy(
            u_t_hbm.at[:, pl.ds(off_of(ru), FW)], ubuf.at[slot], usem)
        pltpu.async_copy(
            i_t_hbm.at[:, pl.ds(off_of(ri), FW)], vbuf.at[slot], isem)

    ruv0 = uidx_vm[pl.ds(0, L)]
    riv0 = iidx_vm[pl.ds(0, L)]
    for j in range(KQ):
        fire(j, ruv0[j], riv0[j])

    def gbody(g, _):
        ruv = uidx_vm[pl.ds(g * L, L)]
        riv = iidx_vm[pl.ds(g * L, L)]
        nbase = jnp.minimum((g + 1) * L, BPW - L)
        rnu = uidx_vm[pl.ds(nbase, L)]
        rni = iidx_vm[pl.ds(nbase, L)]
        for j in range(L):
            b = g * L + j
            slot = lax.rem(b, KQ)
            pltpu.make_async_copy(
                u_t_hbm.at[:, pl.ds(0, FW)], ubuf.at[slot], usem).wait()
            pltpu.make_async_copy(
                i_t_hbm.at[:, pl.ds(0, FW)], vbuf.at[slot], isem).wait()
            ru = ruv[j]
            ri = riv[j]
            slot_v = jnp.full((L,), slot, jnp.int32)
            cu = jnp.full((L,), ru & 127, jnp.int32)
            cv = jnp.full((L,), ri & 127, jnp.int32)
            jv = jnp.full((L,), j, jnp.int32)
            u0 = plsc.load_gather(ubuf, [slot_v, rows0, cu])
            u1 = plsc.load_gather(ubuf, [slot_v, rows0 + L, cu])
            v0 = plsc.load_gather(vbuf, [slot_v, rows0, cv])
            v1 = plsc.load_gather(vbuf, [slot_v, rows0 + L, cv])
            plsc.store_scatter(ustage, [rows0, jv], u0)
            plsc.store_scatter(ustage, [rows0 + L, jv], u1)
            plsc.store_scatter(vstage, [rows0, jv], v0)
            plsc.store_scatter(vstage, [rows0 + L, jv], v1)

            if j + KQ < L:
                nru, nri = ruv[j + KQ], riv[j + KQ]
            else:
                nru, nri = rnu[j + KQ - L], rni[j + KQ - L]

            @pl.when(b + KQ < BPW)
            def _():
                fire(slot, nru, nri)

        acc = jnp.zeros((L,), jnp.float32)
        for d in range(D):
            acc = acc + ustage[d] * vstage[d]
        out_v[pl.ds(g * L, L)] = acc
        return 0

    lax.fori_loop(0, BPW // L, gbody, 0)

    pltpu.sync_copy(out_v, out_hbm.at[pl.ds(base, BPW)])


@jax.jit
def _run(user_idx, item_idx, user_w, item_w):
    mesh = plsc.VectorSubcoreMesh(core_axis_name="c", subcore_axis_name="s")
    k = functools.partial(
        pl.kernel,
        out_type=jax.ShapeDtypeStruct((B,), jnp.float32),
        mesh=mesh,
        compiler_params=pltpu.CompilerParams(
            needs_layout_passes=False, use_tc_tiling_on_sc=True),
        scratch_types=[
            pltpu.VMEM((BPW,), jnp.int32),
            pltpu.VMEM((BPW,), jnp.int32),
            pltpu.VMEM((KQ, D, FW), jnp.float32),
            pltpu.VMEM((KQ, D, FW), jnp.float32),
            pltpu.VMEM((D, L), jnp.float32),
            pltpu.VMEM((D, L), jnp.float32),
            pltpu.VMEM((BPW,), jnp.float32),
            pltpu.SemaphoreType.DMA,
            pltpu.SemaphoreType.DMA,
        ],
    )(_dot_kernel)
    return k(user_idx, item_idx, user_w.T, item_w.T)


def kernel(user_idx, item_idx, user_w, item_w):
    return _run(user_idx, item_idx, user_w, item_w)


# speedup vs baseline: 5.4146x; 1.2210x over previous
import functools

import jax
import jax.numpy as jnp
from jax import lax
from jax.experimental import pallas as pl
from jax.experimental.pallas import tpu as pltpu
from jax.experimental.pallas import tpu_sc as plsc

NC = 2
NS = 16
L = 16
NW = NC * NS

B = 16384
D = 32
BPW = B // NW
NG = 7813
GPW = 245
CAP = 1008
SCAP = 256 * L
KQ = 8


def _gather_kernel(uidx_hbm, iidx_hbm, u_t_hbm, i_t_hbm,
                   urows_hbm, vrows_hbm,
                   idx_vm, sel_r, sel_b, slots_r, slots_b, olist_sm,
                   hist_sm, gbuf, rowstage,
                   gsem, wsem):
    wid = lax.axis_index("s") * NC + lax.axis_index("c")
    rows0 = lax.iota(jnp.int32, L)

    def one_table(idx_hbm, t_hbm, rows_out_hbm):
        pltpu.sync_copy(idx_hbm, idx_vm)

        def zero_hist(t, _):
            hist_sm[t] = 0
            return 0
        lax.fori_loop(0, 256, zero_hist, 0)

        def sel_body(k, cnt):
            chunk = idx_vm[pl.ds(k * L, L)]
            bpos = k * L + rows0
            g = lax.shift_right_logical(chunk, 7)
            own = (g & 31) == wid
            csum = plsc.cumsum(jnp.where(own, 1, 0).astype(jnp.int32))
            dest = cnt + csum - 1
            okm = own & (dest < CAP)
            plsc.store_scatter(sel_r, [dest], chunk, mask=okm)
            plsc.store_scatter(sel_b, [dest], bpos, mask=okm)
            return cnt + csum[L - 1]
        cnt = lax.fori_loop(0, B // L, sel_body, jnp.int32(0))

        def bin_body(k, _):
            chunk = sel_r[pl.ds(k * L, L)]
            bchunk = sel_b[pl.ds(k * L, L)]
            valid = (k * L + rows0) < cnt
            lgv = jnp.where(valid, lax.shift_right_logical(chunk, 12), 255)
            posv = jnp.zeros((L,), jnp.int32)
            for j in range(L):
                lg = lgv[j]
                c = hist_sm[lg]
                hist_sm[lg] = c + 1
                pj = jnp.full((L,), lg * L + c, jnp.int32)
                posv = jnp.where(rows0 == j, pj, posv)
            plsc.store_scatter(slots_r, [posv], chunk, mask=valid)
            plsc.store_scatter(slots_b, [posv], bchunk, mask=valid)
            return 0
        lax.fori_loop(0, (cnt + L - 1) // L, bin_body, 0)

        def ol_body(t, n):
            c = hist_sm[t]

            def put():
                olist_sm[n] = t
                return n + 1
            return jax.lax.cond(c > 0, put, lambda: n)
        ocnt = lax.fori_loop(0, GPW, ol_body, jnp.int32(0))

        def fire(oi):
            gi = olist_sm[oi]
            gcol = (gi * 32 + wid) * 128
            gcol = pl.multiple_of(gcol, 128)
            pltpu.async_copy(t_hbm.at[:, pl.ds(gcol, 128)],
                             gbuf.at[lax.rem(oi, KQ)], gsem)

        def prime(oi, _):
            @pl.when(oi < ocnt)
            def _():
                fire(oi)
            return 0
        lax.fori_loop(0, KQ, prime, 0)

        def grp_body(oi, _):
            slot = lax.rem(oi, KQ)
            pltpu.make_async_copy(t_hbm.at[:, pl.ds(0, 128)],
                                  gbuf.at[slot], gsem).wait()
            gi = olist_sm[oi]
            cg = hist_sm[gi]
            sbase = gi * L

            def member(m, _):
                rmv = plsc.load_gather(slots_r,
                                       [jnp.full((L,), sbase, jnp.int32) + m])
                bmv = plsc.load_gather(slots_b,
                                       [jnp.full((L,), sbase, jnp.int32) + m])
                col = rmv & 127
                sv = jnp.full((L,), slot, jnp.int32)
                u0 = plsc.load_gather(gbuf, [sv, rows0, col])
                u1 = plsc.load_gather(gbuf, [sv, rows0 + L, col])
                rs = lax.rem(m, L)
                rowstage[rs, pl.ds(0, L)] = u0
                rowstage[rs, pl.ds(L, L)] = u1
                b0 = bmv[0] * D
                pltpu.async_copy(rowstage.at[rs],
                                 rows_out_hbm.at[pl.ds(b0, D)], wsem)
                return 0
            lax.fori_loop(0, cg, member, 0)

            def drain(m, _):
                pltpu.make_async_copy(rowstage.at[0],
                                      rows_out_hbm.at[pl.ds(0, D)],
                                      wsem).wait()
                return 0
            lax.fori_loop(0, cg, drain, 0)

            @pl.when(oi + KQ < ocnt)
            def _():
                fire(oi + KQ)
            return 0
        lax.fori_loop(0, ocnt, grp_body, 0)

    one_table(uidx_hbm, u_t_hbm, urows_hbm)
    one_table(iidx_hbm, i_t_hbm, vrows_hbm)


def _dot_kernel(urows_hbm, vrows_hbm, out_hbm, u_vm, v_vm, out_v, sem1, sem2):
    wid = lax.axis_index("s") * NC + lax.axis_index("c")
    base = wid * BPW
    c1 = pltpu.async_copy(urows_hbm.at[pl.ds(base * D, BPW * D)], u_vm, sem1)
    c2 = pltpu.async_copy(vrows_hbm.at[pl.ds(base * D, BPW * D)], v_vm, sem2)
    c1.wait()
    c2.wait()
    rows0 = lax.iota(jnp.int32, L)

    def gbody(g, _):
        fbase = g * L * D
        acc = jnp.zeros((L,), jnp.float32)
        for d in range(D):
            fidx = fbase + rows0 * D + d
            u = plsc.load_gather(u_vm, [fidx])
            v = plsc.load_gather(v_vm, [fidx])
            acc = acc + u * v
        out_v[pl.ds(g * L, L)] = acc
        return 0
    lax.fori_loop(0, BPW // L, gbody, 0)
    pltpu.sync_copy(out_v, out_hbm.at[pl.ds(base, BPW)])


@jax.jit
def _run(user_idx, item_idx, user_w, item_w):
    mesh = plsc.VectorSubcoreMesh(core_axis_name="c", subcore_axis_name="s")
    k1 = functools.partial(
        pl.kernel,
        out_type=(jax.ShapeDtypeStruct((B * D,), jnp.float32),
                  jax.ShapeDtypeStruct((B * D,), jnp.float32)),
        mesh=mesh,
        compiler_params=pltpu.CompilerParams(
            needs_layout_passes=False, use_tc_tiling_on_sc=True),
        scratch_types=[
            pltpu.VMEM((B,), jnp.int32),
            pltpu.VMEM((CAP,), jnp.int32),
            pltpu.VMEM((CAP,), jnp.int32),
            pltpu.VMEM((SCAP,), jnp.int32),
            pltpu.VMEM((SCAP,), jnp.int32),
            pltpu.SMEM((256,), jnp.int32),
            pltpu.SMEM((256,), jnp.int32),
            pltpu.VMEM((KQ, D, 128), jnp.float32),
            pltpu.VMEM((L, D), jnp.float32),
            pltpu.SemaphoreType.DMA,
            pltpu.SemaphoreType.DMA,
        ],
    )(_gather_kernel)
    urows, vrows = k1(user_idx, item_idx, user_w.T, item_w.T)

    k2 = functools.partial(
        pl.kernel,
        out_type=jax.ShapeDtypeStruct((B,), jnp.float32),
        mesh=mesh,
        compiler_params=pltpu.CompilerParams(
            needs_layout_passes=False, use_tc_tiling_on_sc=True),
        scratch_types=[
            pltpu.VMEM((BPW * D,), jnp.float32),
            pltpu.VMEM((BPW * D,), jnp.float32),
            pltpu.VMEM((BPW,), jnp.float32),
            pltpu.SemaphoreType.DMA,
            pltpu.SemaphoreType.DMA,
        ],
    )(_dot_kernel)
    return k2(urows, vrows)


def kernel(user_idx, item_idx, user_w, item_w):
    return _run(user_idx, item_idx, user_w, item_w)


# speedup vs baseline: 5.6057x; 1.0353x over previous
import functools

import jax
import jax.numpy as jnp
from jax import lax
from jax.experimental import pallas as pl
from jax.experimental.pallas import tpu as pltpu
from jax.experimental.pallas import tpu_sc as plsc

NC = 2
NS = 16
L = 16
NW = NC * NS

B = 16384
D = 32
BPW = B // NW
NG = 7813
GPW = 245
CAP = 1008
SCAP = 256 * L
KQ = 12
RS = 32


def _gather_kernel(uidx_hbm, iidx_hbm, u_t_hbm, i_t_hbm,
                   urows_hbm, vrows_hbm,
                   idx_vm, sel_r, sel_b, slots_r, slots_b, olist_sm,
                   hist_sm, gbuf, rowstage,
                   gsem, wsem):
    wid = lax.axis_index("s") * NC + lax.axis_index("c")
    rows0 = lax.iota(jnp.int32, L)

    def one_table(idx_hbm, t_hbm, rows_out_hbm):
        pltpu.sync_copy(idx_hbm, idx_vm)

        def zero_hist(t, _):
            hist_sm[t] = 0
            return 0
        lax.fori_loop(0, 256, zero_hist, 0)

        def sel_body(k, cnt):
            chunk = idx_vm[pl.ds(k * L, L)]
            bpos = k * L + rows0
            g = lax.shift_right_logical(chunk, 7)
            own = (g & 31) == wid
            csum = plsc.cumsum(jnp.where(own, 1, 0).astype(jnp.int32))
            dest = cnt + csum - 1
            okm = own & (dest < CAP)
            plsc.store_scatter(sel_r, [dest], chunk, mask=okm)
            plsc.store_scatter(sel_b, [dest], bpos, mask=okm)
            return cnt + csum[L - 1]
        cnt = lax.fori_loop(0, B // L, sel_body, jnp.int32(0))

        def bin_body(k, _):
            chunk = sel_r[pl.ds(k * L, L)]
            bchunk = sel_b[pl.ds(k * L, L)]
            valid = (k * L + rows0) < cnt
            lgv = jnp.where(valid, lax.shift_right_logical(chunk, 12), 255)
            posv = jnp.zeros((L,), jnp.int32)
            for j in range(L):
                lg = lgv[j]
                c = hist_sm[lg]
                hist_sm[lg] = c + 1
                pj = jnp.full((L,), lg * L + c, jnp.int32)
                posv = jnp.where(rows0 == j, pj, posv)
            plsc.store_scatter(slots_r, [posv], chunk, mask=valid)
            plsc.store_scatter(slots_b, [posv], bchunk, mask=valid)
            return 0
        lax.fori_loop(0, (cnt + L - 1) // L, bin_body, 0)

        def ol_body(t, n):
            c = hist_sm[t]

            def put():
                olist_sm[n] = t
                return n + 1
            return jax.lax.cond(c > 0, put, lambda: n)
        ocnt = lax.fori_loop(0, GPW, ol_body, jnp.int32(0))

        def fire(oi):
            gi = olist_sm[oi]
            gcol = (gi * 32 + wid) * 128
            gcol = pl.multiple_of(gcol, 128)
            pltpu.async_copy(t_hbm.at[:, pl.ds(gcol, 128)],
                             gbuf.at[lax.rem(oi, KQ)], gsem)

        def prime(oi, _):
            @pl.when(oi < ocnt)
            def _():
                fire(oi)
            return 0
        lax.fori_loop(0, KQ, prime, 0)

        def grp_body(oi, gwc):
            slot = lax.rem(oi, KQ)
            pltpu.make_async_copy(t_hbm.at[:, pl.ds(0, 128)],
                                  gbuf.at[slot], gsem).wait()
            gi = olist_sm[oi]
            cg = hist_sm[gi]
            sbase = gi * L

            @pl.when(oi + KQ < ocnt)
            def _():
                fire(oi + KQ)

            def member(m, _):
                w = gwc + m

                @pl.when(w >= RS)
                def _():
                    pltpu.make_async_copy(rowstage.at[0],
                                          rows_out_hbm.at[pl.ds(0, D)],
                                          wsem).wait()
                rmv = plsc.load_gather(slots_r,
                                       [jnp.full((L,), sbase, jnp.int32) + m])
                bmv = plsc.load_gather(slots_b,
                                       [jnp.full((L,), sbase, jnp.int32) + m])
                col = rmv & 127
                sv = jnp.full((L,), slot, jnp.int32)
                u0 = plsc.load_gather(gbuf, [sv, rows0, col])
                u1 = plsc.load_gather(gbuf, [sv, rows0 + L, col])
                rs = lax.rem(w, RS)
                rowstage[rs, pl.ds(0, L)] = u0
                rowstage[rs, pl.ds(L, L)] = u1
                b0 = bmv[0] * D
                pltpu.async_copy(rowstage.at[rs],
                                 rows_out_hbm.at[pl.ds(b0, D)], wsem)
                return 0
            lax.fori_loop(0, cg, member, 0)
            return gwc + cg
        wcnt = lax.fori_loop(0, ocnt, grp_body, jnp.int32(0))

        def final_drain(m, _):
            pltpu.make_async_copy(rowstage.at[0],
                                  rows_out_hbm.at[pl.ds(0, D)], wsem).wait()
            return 0
        lax.fori_loop(0, jnp.minimum(wcnt, RS), final_drain, 0)

    one_table(uidx_hbm, u_t_hbm, urows_hbm)
    one_table(iidx_hbm, i_t_hbm, vrows_hbm)


def _dot_kernel(urows_hbm, vrows_hbm, out_hbm, u_vm, v_vm, out_v, sem1, sem2):
    wid = lax.axis_index("s") * NC + lax.axis_index("c")
    base = wid * BPW
    c1 = pltpu.async_copy(urows_hbm.at[pl.ds(base * D, BPW * D)], u_vm, sem1)
    c2 = pltpu.async_copy(vrows_hbm.at[pl.ds(base * D, BPW * D)], v_vm, sem2)
    c1.wait()
    c2.wait()
    rows0 = lax.iota(jnp.int32, L)

    def gbody(g, _):
        fbase = g * L * D
        acc = jnp.zeros((L,), jnp.float32)
        for d in range(D):
            fidx = fbase + rows0 * D + d
            u = plsc.load_gather(u_vm, [fidx])
            v = plsc.load_gather(v_vm, [fidx])
            acc = acc + u * v
        out_v[pl.ds(g * L, L)] = acc
        return 0
    lax.fori_loop(0, BPW // L, gbody, 0)
    pltpu.sync_copy(out_v, out_hbm.at[pl.ds(base, BPW)])


@jax.jit
def _run(user_idx, item_idx, user_w, item_w):
    mesh = plsc.VectorSubcoreMesh(core_axis_name="c", subcore_axis_name="s")
    k1 = functools.partial(
        pl.kernel,
        out_type=(jax.ShapeDtypeStruct((B * D,), jnp.float32),
                  jax.ShapeDtypeStruct((B * D,), jnp.float32)),
        mesh=mesh,
        compiler_params=pltpu.CompilerParams(
            needs_layout_passes=False, use_tc_tiling_on_sc=True),
        scratch_types=[
            pltpu.VMEM((B,), jnp.int32),
            pltpu.VMEM((CAP,), jnp.int32),
            pltpu.VMEM((CAP,), jnp.int32),
            pltpu.VMEM((SCAP,), jnp.int32),
            pltpu.VMEM((SCAP,), jnp.int32),
            pltpu.SMEM((256,), jnp.int32),
            pltpu.SMEM((256,), jnp.int32),
            pltpu.VMEM((KQ, D, 128), jnp.float32),
            pltpu.VMEM((RS, D), jnp.float32),
            pltpu.SemaphoreType.DMA,
            pltpu.SemaphoreType.DMA,
        ],
    )(_gather_kernel)
    urows, vrows = k1(user_idx, item_idx, user_w.T, item_w.T)

    k2 = functools.partial(
        pl.kernel,
        out_type=jax.ShapeDtypeStruct((B,), jnp.float32),
        mesh=mesh,
        compiler_params=pltpu.CompilerParams(
            needs_layout_passes=False, use_tc_tiling_on_sc=True),
        scratch_types=[
            pltpu.VMEM((BPW * D,), jnp.float32),
            pltpu.VMEM((BPW * D,), jnp.float32),
            pltpu.VMEM((BPW,), jnp.float32),
            pltpu.SemaphoreType.DMA,
            pltpu.SemaphoreType.DMA,
        ],
    )(_dot_kernel)
    return k2(urows, vrows)


def kernel(user_idx, item_idx, user_w, item_w):
    return _run(user_idx, item_idx, user_w, item_w)


# speedup vs baseline: 6.2237x; 1.1102x over previous
import functools

import jax
import jax.numpy as jnp
from jax import lax
from jax.experimental import pallas as pl
from jax.experimental.pallas import tpu as pltpu
from jax.experimental.pallas import tpu_sc as plsc

NC = 2
NS = 16
L = 16
NW = NC * NS

B = 16384
D = 32
BPW = B // NW
NG = 7813
GPW = 245
CAP = 1008
SCAP = 256 * L
KQ = 12
RS = 32


def _gather_kernel(uidx_hbm, iidx_hbm, u_t_hbm, i_t_hbm,
                   urows_hbm, vrows_hbm,
                   idx_vm, sel_r, sel_b, slots_r, slots_b, olist_sm,
                   hist_sm, gbuf, rowstage,
                   gsem, wsem):
    wid = lax.axis_index("s") * NC + lax.axis_index("c")
    rows0 = lax.iota(jnp.int32, L)

    def one_table(idx_hbm, t_hbm, rows_out_hbm):
        pltpu.sync_copy(idx_hbm, idx_vm)

        def zero_hist(t, _):
            hist_sm[t] = 0
            return 0
        lax.fori_loop(0, 256, zero_hist, 0)

        def sel_body(q, cnt):
            chunks, csums = [], []
            for t in range(4):
                chunk = idx_vm[pl.ds((q * 4 + t) * L, L)]
                g = lax.shift_right_logical(chunk, 7)
                own = (g & 31) == wid
                csums.append(plsc.cumsum(jnp.where(own, 1, 0)
                                         .astype(jnp.int32)))
                chunks.append((chunk, own))
            for t in range(4):
                chunk, own = chunks[t]
                bpos = (q * 4 + t) * L + rows0
                dest = cnt + csums[t] - 1
                okm = own & (dest < CAP)
                plsc.store_scatter(sel_r, [dest], chunk, mask=okm)
                plsc.store_scatter(sel_b, [dest], bpos, mask=okm)
                cnt = cnt + csums[t][L - 1]
            return cnt
        cnt = lax.fori_loop(0, B // L // 4, sel_body, jnp.int32(0))

        def bin_body(k, _):
            chunk = sel_r[pl.ds(k * L, L)]
            bchunk = sel_b[pl.ds(k * L, L)]
            valid = (k * L + rows0) < cnt
            lgv = jnp.where(valid, lax.shift_right_logical(chunk, 12), 255)
            posv = jnp.zeros((L,), jnp.int32)
            for j in range(L):
                lg = lgv[j]
                c = hist_sm[lg]
                hist_sm[lg] = c + 1
                pj = jnp.full((L,), lg * L + c, jnp.int32)
                posv = jnp.where(rows0 == j, pj, posv)
            plsc.store_scatter(slots_r, [posv], chunk, mask=valid)
            plsc.store_scatter(slots_b, [posv], bchunk, mask=valid)
            return 0
        lax.fori_loop(0, (cnt + L - 1) // L, bin_body, 0)

        def ol_body(t, n):
            c = hist_sm[t]

            def put():
                olist_sm[n] = t
                return n + 1
            return jax.lax.cond(c > 0, put, lambda: n)
        ocnt = lax.fori_loop(0, GPW, ol_body, jnp.int32(0))

        def fire(oi):
            gi = olist_sm[oi]
            gcol = (gi * 32 + wid) * 128
            gcol = pl.multiple_of(gcol, 128)
            pltpu.async_copy(t_hbm.at[:, pl.ds(gcol, 128)],
                             gbuf.at[lax.rem(oi, KQ)], gsem)

        def prime(oi, _):
            @pl.when(oi < ocnt)
            def _():
                fire(oi)
            return 0
        lax.fori_loop(0, KQ, prime, 0)

        def grp_body(oi, gwc):
            slot = lax.rem(oi, KQ)
            pltpu.make_async_copy(t_hbm.at[:, pl.ds(0, 128)],
                                  gbuf.at[slot], gsem).wait()
            gi = olist_sm[oi]
            cg = hist_sm[gi]
            sbase = gi * L

            @pl.when(oi + KQ < ocnt)
            def _():
                fire(oi + KQ)

            def member(m, _):
                w = gwc + m

                @pl.when(w >= RS)
                def _():
                    pltpu.make_async_copy(rowstage.at[0],
                                          rows_out_hbm.at[pl.ds(0, D)],
                                          wsem).wait()
                rmv = plsc.load_gather(slots_r,
                                       [jnp.full((L,), sbase, jnp.int32) + m])
                bmv = plsc.load_gather(slots_b,
                                       [jnp.full((L,), sbase, jnp.int32) + m])
                col = rmv & 127
                sv = jnp.full((L,), slot, jnp.int32)
                u0 = plsc.load_gather(gbuf, [sv, rows0, col])
                u1 = plsc.load_gather(gbuf, [sv, rows0 + L, col])
                rs = lax.rem(w, RS)
                rowstage[rs, pl.ds(0, L)] = u0
                rowstage[rs, pl.ds(L, L)] = u1
                b0 = bmv[0] * D
                pltpu.async_copy(rowstage.at[rs],
                                 rows_out_hbm.at[pl.ds(b0, D)], wsem)
                return 0
            lax.fori_loop(0, cg, member, 0)
            return gwc + cg
        wcnt = lax.fori_loop(0, ocnt, grp_body, jnp.int32(0))

        def final_drain(m, _):
            pltpu.make_async_copy(rowstage.at[0],
                                  rows_out_hbm.at[pl.ds(0, D)], wsem).wait()
            return 0
        lax.fori_loop(0, jnp.minimum(wcnt, RS), final_drain, 0)

    one_table(uidx_hbm, u_t_hbm, urows_hbm)
    one_table(iidx_hbm, i_t_hbm, vrows_hbm)


def _dot_kernel(urows_hbm, vrows_hbm, out_hbm, u_vm, v_vm, out_v, sem1, sem2):
    wid = lax.axis_index("s") * NC + lax.axis_index("c")
    base = wid * BPW
    c1 = pltpu.async_copy(urows_hbm.at[pl.ds(base * D, BPW * D)], u_vm, sem1)
    c2 = pltpu.async_copy(vrows_hbm.at[pl.ds(base * D, BPW * D)], v_vm, sem2)
    c1.wait()
    c2.wait()
    rows0 = lax.iota(jnp.int32, L)

    def gbody(g, _):
        fbase = g * L * D
        acc = jnp.zeros((L,), jnp.float32)
        for d in range(D):
            fidx = fbase + rows0 * D + d
            u = plsc.load_gather(u_vm, [fidx])
            v = plsc.load_gather(v_vm, [fidx])
            acc = acc + u * v
        out_v[pl.ds(g * L, L)] = acc
        return 0
    lax.fori_loop(0, BPW // L, gbody, 0)
    pltpu.sync_copy(out_v, out_hbm.at[pl.ds(base, BPW)])


@jax.jit
def _run(user_idx, item_idx, user_w, item_w):
    mesh = plsc.VectorSubcoreMesh(core_axis_name="c", subcore_axis_name="s")
    k1 = functools.partial(
        pl.kernel,
        out_type=(jax.ShapeDtypeStruct((B * D,), jnp.float32),
                  jax.ShapeDtypeStruct((B * D,), jnp.float32)),
        mesh=mesh,
        compiler_params=pltpu.CompilerParams(
            needs_layout_passes=False, use_tc_tiling_on_sc=True),
        scratch_types=[
            pltpu.VMEM((B,), jnp.int32),
            pltpu.VMEM((CAP,), jnp.int32),
            pltpu.VMEM((CAP,), jnp.int32),
            pltpu.VMEM((SCAP,), jnp.int32),
            pltpu.VMEM((SCAP,), jnp.int32),
            pltpu.SMEM((256,), jnp.int32),
            pltpu.SMEM((256,), jnp.int32),
            pltpu.VMEM((KQ, D, 128), jnp.float32),
            pltpu.VMEM((RS, D), jnp.float32),
            pltpu.SemaphoreType.DMA,
            pltpu.SemaphoreType.DMA,
        ],
    )(_gather_kernel)
    urows, vrows = k1(user_idx, item_idx, user_w.T, item_w.T)

    k2 = functools.partial(
        pl.kernel,
        out_type=jax.ShapeDtypeStruct((B,), jnp.float32),
        mesh=mesh,
        compiler_params=pltpu.CompilerParams(
            needs_layout_passes=False, use_tc_tiling_on_sc=True),
        scratch_types=[
            pltpu.VMEM((BPW * D,), jnp.float32),
            pltpu.VMEM((BPW * D,), jnp.float32),
            pltpu.VMEM((BPW,), jnp.float32),
            pltpu.SemaphoreType.DMA,
            pltpu.SemaphoreType.DMA,
        ],
    )(_dot_kernel)
    return k2(urows, vrows)


def kernel(user_idx, item_idx, user_w, item_w):
    return _run(user_idx, item_idx, user_w, item_w)


# speedup vs baseline: 6.4305x; 1.0332x over previous
import functools

import jax
import jax.numpy as jnp
from jax import lax
from jax.experimental import pallas as pl
from jax.experimental.pallas import tpu as pltpu
from jax.experimental.pallas import tpu_sc as plsc

NC = 2
NS = 16
L = 16
NW = NC * NS

B = 16384
D = 32
BPW = B // NW
NG = 7813
GPW = 245
CAP = 1008
SCAP = 256 * L
KQ = 16
RS = 32


def _gather_kernel(uidx_hbm, iidx_hbm, u_t_hbm, i_t_hbm,
                   urows_hbm, vrows_hbm,
                   idx_vm, sel_r, sel_b, slots_r, slots_b, olist_sm,
                   hist_sm, gbuf, rowstage,
                   gsem, wsem):
    wid = lax.axis_index("s") * NC + lax.axis_index("c")
    rows0 = lax.iota(jnp.int32, L)

    def one_table(idx_hbm, t_hbm, rows_out_hbm):
        pltpu.sync_copy(idx_hbm, idx_vm)

        def zero_hist(t, _):
            hist_sm[t] = 0
            return 0
        lax.fori_loop(0, 256, zero_hist, 0)

        def sel_body(q, cnt):
            chunks, csums = [], []
            for t in range(8):
                chunk = idx_vm[pl.ds((q * 8 + t) * L, L)]
                g = lax.shift_right_logical(chunk, 7)
                own = (g & 31) == wid
                csums.append(plsc.cumsum(jnp.where(own, 1, 0)
                                         .astype(jnp.int32)))
                chunks.append((chunk, own))
            for t in range(8):
                chunk, own = chunks[t]
                bpos = (q * 8 + t) * L + rows0
                dest = cnt + csums[t] - 1
                okm = own & (dest < CAP)
                plsc.store_scatter(sel_r, [dest], chunk, mask=okm)
                plsc.store_scatter(sel_b, [dest], bpos, mask=okm)
                cnt = cnt + csums[t][L - 1]
            return cnt
        cnt = lax.fori_loop(0, B // L // 8, sel_body, jnp.int32(0))

        def bin_body(k, _):
            chunk = sel_r[pl.ds(k * L, L)]
            bchunk = sel_b[pl.ds(k * L, L)]
            valid = (k * L + rows0) < cnt
            lgv = jnp.where(valid, lax.shift_right_logical(chunk, 12), 255)
            posv = jnp.zeros((L,), jnp.int32)
            for j in range(L):
                lg = lgv[j]
                c = hist_sm[lg]
                hist_sm[lg] = c + 1
                pj = jnp.full((L,), lg * L + c, jnp.int32)
                posv = jnp.where(rows0 == j, pj, posv)
            plsc.store_scatter(slots_r, [posv], chunk, mask=valid)
            plsc.store_scatter(slots_b, [posv], bchunk, mask=valid)
            return 0
        lax.fori_loop(0, (cnt + L - 1) // L, bin_body, 0)

        def ol_body(t, n):
            c = hist_sm[t]

            def put():
                olist_sm[n] = t
                return n + 1
            return jax.lax.cond(c > 0, put, lambda: n)
        ocnt = lax.fori_loop(0, GPW, ol_body, jnp.int32(0))

        def fire(oi):
            gi = olist_sm[oi]
            gcol = (gi * 32 + wid) * 128
            gcol = pl.multiple_of(gcol, 128)
            pltpu.async_copy(t_hbm.at[:, pl.ds(gcol, 128)],
                             gbuf.at[lax.rem(oi, KQ)], gsem)

        def prime(oi, _):
            @pl.when(oi < ocnt)
            def _():
                fire(oi)
            return 0
        lax.fori_loop(0, KQ, prime, 0)

        def grp_body(oi, gwc):
            slot = lax.rem(oi, KQ)
            pltpu.make_async_copy(t_hbm.at[:, pl.ds(0, 128)],
                                  gbuf.at[slot], gsem).wait()
            gi = olist_sm[oi]
            cg = hist_sm[gi]
            sbase = gi * L

            @pl.when(oi + KQ < ocnt)
            def _():
                fire(oi + KQ)

            def member(m, _):
                w = gwc + m

                @pl.when(w >= RS)
                def _():
                    pltpu.make_async_copy(rowstage.at[0],
                                          rows_out_hbm.at[pl.ds(0, D)],
                                          wsem).wait()
                rmv = plsc.load_gather(slots_r,
                                       [jnp.full((L,), sbase, jnp.int32) + m])
                bmv = plsc.load_gather(slots_b,
                                       [jnp.full((L,), sbase, jnp.int32) + m])
                col = rmv & 127
                sv = jnp.full((L,), slot, jnp.int32)
                u0 = plsc.load_gather(gbuf, [sv, rows0, col])
                u1 = plsc.load_gather(gbuf, [sv, rows0 + L, col])
                rs = lax.rem(w, RS)
                rowstage[rs, pl.ds(0, L)] = u0
                rowstage[rs, pl.ds(L, L)] = u1
                b0 = bmv[0] * D
                pltpu.async_copy(rowstage.at[rs],
                                 rows_out_hbm.at[pl.ds(b0, D)], wsem)
                return 0
            lax.fori_loop(0, cg, member, 0)
            return gwc + cg
        wcnt = lax.fori_loop(0, ocnt, grp_body, jnp.int32(0))

        def final_drain(m, _):
            pltpu.make_async_copy(rowstage.at[0],
                                  rows_out_hbm.at[pl.ds(0, D)], wsem).wait()
            return 0
        lax.fori_loop(0, jnp.minimum(wcnt, RS), final_drain, 0)

    one_table(uidx_hbm, u_t_hbm, urows_hbm)
    one_table(iidx_hbm, i_t_hbm, vrows_hbm)


def _dot_kernel(urows_hbm, vrows_hbm, out_hbm, u_vm, v_vm, out_v, sem1, sem2):
    wid = lax.axis_index("s") * NC + lax.axis_index("c")
    base = wid * BPW
    c1 = pltpu.async_copy(urows_hbm.at[pl.ds(base * D, BPW * D)], u_vm, sem1)
    c2 = pltpu.async_copy(vrows_hbm.at[pl.ds(base * D, BPW * D)], v_vm, sem2)
    c1.wait()
    c2.wait()
    rows0 = lax.iota(jnp.int32, L)

    def gbody(g, _):
        fbase = g * L * D
        acc = jnp.zeros((L,), jnp.float32)
        for d in range(D):
            fidx = fbase + rows0 * D + d
            u = plsc.load_gather(u_vm, [fidx])
            v = plsc.load_gather(v_vm, [fidx])
            acc = acc + u * v
        out_v[pl.ds(g * L, L)] = acc
        return 0
    lax.fori_loop(0, BPW // L, gbody, 0)
    pltpu.sync_copy(out_v, out_hbm.at[pl.ds(base, BPW)])


@jax.jit
def _run(user_idx, item_idx, user_w, item_w):
    mesh = plsc.VectorSubcoreMesh(core_axis_name="c", subcore_axis_name="s")
    k1 = functools.partial(
        pl.kernel,
        out_type=(jax.ShapeDtypeStruct((B * D,), jnp.float32),
                  jax.ShapeDtypeStruct((B * D,), jnp.float32)),
        mesh=mesh,
        compiler_params=pltpu.CompilerParams(
            needs_layout_passes=False, use_tc_tiling_on_sc=True),
        scratch_types=[
            pltpu.VMEM((B,), jnp.int32),
            pltpu.VMEM((CAP,), jnp.int32),
            pltpu.VMEM((CAP,), jnp.int32),
            pltpu.VMEM((SCAP,), jnp.int32),
            pltpu.VMEM((SCAP,), jnp.int32),
            pltpu.SMEM((256,), jnp.int32),
            pltpu.SMEM((256,), jnp.int32),
            pltpu.VMEM((KQ, D, 128), jnp.float32),
            pltpu.VMEM((RS, D), jnp.float32),
            pltpu.SemaphoreType.DMA,
            pltpu.SemaphoreType.DMA,
        ],
    )(_gather_kernel)
    urows, vrows = k1(user_idx, item_idx, user_w.T, item_w.T)

    k2 = functools.partial(
        pl.kernel,
        out_type=jax.ShapeDtypeStruct((B,), jnp.float32),
        mesh=mesh,
        compiler_params=pltpu.CompilerParams(
            needs_layout_passes=False, use_tc_tiling_on_sc=True),
        scratch_types=[
            pltpu.VMEM((BPW * D,), jnp.float32),
            pltpu.VMEM((BPW * D,), jnp.float32),
            pltpu.VMEM((BPW,), jnp.float32),
            pltpu.SemaphoreType.DMA,
            pltpu.SemaphoreType.DMA,
        ],
    )(_dot_kernel)
    return k2(urows, vrows)


def kernel(user_idx, item_idx, user_w, item_w):
    return _run(user_idx, item_idx, user_w, item_w)


# speedup vs baseline: 6.8180x; 1.0603x over previous
import functools

import jax
import jax.numpy as jnp
from jax import lax
from jax.experimental import pallas as pl
from jax.experimental.pallas import tpu as pltpu
from jax.experimental.pallas import tpu_sc as plsc

NC = 2
NS = 16
L = 16
NW = NC * NS

B = 16384
D = 32
BPW = B // NW
NG = 7813
GPW = 245
CAP = 1008
SCAP = 256 * L
KQ = 14
RS = 32


def _gather_kernel(uidx_hbm, iidx_hbm, u_t_hbm, i_t_hbm,
                   urows_hbm, vrows_hbm,
                   idx_vm, idx2_vm, sel_r, sel_b, sel2_r, sel2_b,
                   slots_r, slots_b, slots2_r, slots2_b,
                   olist_sm, olist2_sm, hist_sm, hist2_sm, gbuf, rowstage,
                   gsem, wsem):
    wid = lax.axis_index("s") * NC + lax.axis_index("c")
    rows0 = lax.iota(jnp.int32, L)
    SI = B // L // 8

    pltpu.sync_copy(uidx_hbm, idx_vm)
    pltpu.sync_copy(iidx_hbm, idx2_vm)

    def zero_hists(t, _):
        hist_sm[t] = 0
        hist2_sm[t] = 0
        return 0
    lax.fori_loop(0, 256, zero_hists, 0)

    def sel_iter(q, cnt, src_vm, dst_r, dst_b):
        chunks, csums = [], []
        for t in range(8):
            chunk = src_vm[pl.ds((q * 8 + t) * L, L)]
            g = lax.shift_right_logical(chunk, 7)
            own = (g & 31) == wid
            csums.append(plsc.cumsum(jnp.where(own, 1, 0).astype(jnp.int32)))
            chunks.append((chunk, own))
        for t in range(8):
            chunk, own = chunks[t]
            bpos = (q * 8 + t) * L + rows0
            dest = cnt + csums[t] - 1
            okm = own & (dest < CAP)
            plsc.store_scatter(dst_r, [dest], chunk, mask=okm)
            plsc.store_scatter(dst_b, [dest], bpos, mask=okm)
            cnt = cnt + csums[t][L - 1]
        return cnt

    def bin_iter(k, cnt, src_r, src_b, dst_r, dst_b, hist):
        chunk = src_r[pl.ds(k * L, L)]
        bchunk = src_b[pl.ds(k * L, L)]
        valid = (k * L + rows0) < cnt
        lgv = jnp.where(valid, lax.shift_right_logical(chunk, 12), 255)
        posv = jnp.zeros((L,), jnp.int32)
        for j in range(L):
            lg = lgv[j]
            c = hist[lg]
            hist[lg] = c + 1
            pj = jnp.full((L,), lg * L + c, jnp.int32)
            posv = jnp.where(rows0 == j, pj, posv)
        plsc.store_scatter(dst_r, [posv], chunk, mask=valid)
        plsc.store_scatter(dst_b, [posv], bchunk, mask=valid)

    def olist_build(hist, olist):
        def ol_body(t, n):
            c = hist[t]

            def put():
                olist[n] = t
                return n + 1
            return jax.lax.cond(c > 0, put, lambda: n)
        return lax.fori_loop(0, GPW, ol_body, jnp.int32(0))

    def fire(oi, t_hbm, olist):
        gi = olist[oi]
        gcol = (gi * 32 + wid) * 128
        gcol = pl.multiple_of(gcol, 128)
        pltpu.async_copy(t_hbm.at[:, pl.ds(gcol, 128)],
                         gbuf.at[lax.rem(oi, KQ)], gsem)

    def group_work(oi, gwc, t_hbm, rows_out_hbm, olist, hist, ocnt):
        slot = lax.rem(oi, KQ)
        pltpu.make_async_copy(t_hbm.at[:, pl.ds(0, 128)],
                              gbuf.at[slot], gsem).wait()
        gi = olist[oi]
        cg = hist[gi]
        sbase = gi * L

        @pl.when(oi + KQ < ocnt)
        def _():
            fire(oi + KQ, t_hbm, olist)

        def member(m, _):
            w = gwc + m

            @pl.when(w >= RS)
            def _():
                pltpu.make_async_copy(rowstage.at[0],
                                      rows_out_hbm.at[pl.ds(0, D)],
                                      wsem).wait()
            rmv = plsc.load_gather(slots_r if rows_out_hbm is urows_hbm
                                   else slots2_r,
                                   [jnp.full((L,), sbase, jnp.int32) + m])
            bmv = plsc.load_gather(slots_b if rows_out_hbm is urows_hbm
                                   else slots2_b,
                                   [jnp.full((L,), sbase, jnp.int32) + m])
            col = rmv & 127
            sv = jnp.full((L,), slot, jnp.int32)
            u0 = plsc.load_gather(gbuf, [sv, rows0, col])
            u1 = plsc.load_gather(gbuf, [sv, rows0 + L, col])
            rs = lax.rem(w, RS)
            rowstage[rs, pl.ds(0, L)] = u0
            rowstage[rs, pl.ds(L, L)] = u1
            b0 = bmv[0] * D
            pltpu.async_copy(rowstage.at[rs],
                             rows_out_hbm.at[pl.ds(b0, D)], wsem)
            return 0
        lax.fori_loop(0, cg, member, 0)
        return gwc + cg

    def final_drain(rows_out_hbm, wcnt):
        def fd(m, _):
            pltpu.make_async_copy(rowstage.at[0],
                                  rows_out_hbm.at[pl.ds(0, D)], wsem).wait()
            return 0
        lax.fori_loop(0, jnp.minimum(wcnt, RS), fd, 0)

    def sel_u(q, cnt):
        return sel_iter(q, cnt, idx_vm, sel_r, sel_b)
    cnt_u = lax.fori_loop(0, SI, sel_u, jnp.int32(0))

    def bin_u(k, _):
        bin_iter(k, cnt_u, sel_r, sel_b, slots_r, slots_b, hist_sm)
        return 0
    lax.fori_loop(0, (cnt_u + L - 1) // L, bin_u, 0)
    ocnt_u = olist_build(hist_sm, olist_sm)

    def prime_u(oi, _):
        @pl.when(oi < ocnt_u)
        def _():
            fire(oi, u_t_hbm, olist_sm)
        return 0
    lax.fori_loop(0, KQ, prime_u, 0)

    NI = SI + 64

    def walk_u(oi, carry):
        gwc, cnt_i = carry
        gwc = jax.lax.cond(
            oi < ocnt_u,
            lambda: group_work(oi, gwc, u_t_hbm, urows_hbm,
                               olist_sm, hist_sm, ocnt_u),
            lambda: gwc)
        cnt_i = jax.lax.cond(
            oi < SI,
            lambda: sel_iter(oi, cnt_i, idx2_vm, sel2_r, sel2_b),
            lambda: cnt_i)

        @pl.when((oi >= SI) & (oi - SI < (cnt_i + L - 1) // L))
        def _():
            bin_iter(oi - SI, cnt_i, sel2_r, sel2_b,
                     slots2_r, slots2_b, hist2_sm)
        return (gwc, cnt_i)

    loop_n = jnp.maximum(ocnt_u, NI)
    wcnt_u, _cnt_i = lax.fori_loop(0, loop_n, walk_u,
                                   (jnp.int32(0), jnp.int32(0)))
    final_drain(urows_hbm, wcnt_u)

    ocnt_i = olist_build(hist2_sm, olist2_sm)

    def prime_i(oi, _):
        @pl.when(oi < ocnt_i)
        def _():
            fire(oi, i_t_hbm, olist2_sm)
        return 0
    lax.fori_loop(0, KQ, prime_i, 0)

    def walk_i(oi, gwc):
        return group_work(oi, gwc, i_t_hbm, vrows_hbm,
                          olist2_sm, hist2_sm, ocnt_i)
    wcnt_i = lax.fori_loop(0, ocnt_i, walk_i, jnp.int32(0))
    final_drain(vrows_hbm, wcnt_i)


def _dot_kernel(urows_hbm, vrows_hbm, out_hbm, u_vm, v_vm, out_v, sem1, sem2):
    wid = lax.axis_index("s") * NC + lax.axis_index("c")
    base = wid * BPW
    c1 = pltpu.async_copy(urows_hbm.at[pl.ds(base * D, BPW * D)], u_vm, sem1)
    c2 = pltpu.async_copy(vrows_hbm.at[pl.ds(base * D, BPW * D)], v_vm, sem2)
    c1.wait()
    c2.wait()
    rows0 = lax.iota(jnp.int32, L)

    def gbody(g, _):
        fbase = g * L * D
        acc = jnp.zeros((L,), jnp.float32)
        for d in range(D):
            fidx = fbase + rows0 * D + d
            u = plsc.load_gather(u_vm, [fidx])
            v = plsc.load_gather(v_vm, [fidx])
            acc = acc + u * v
        out_v[pl.ds(g * L, L)] = acc
        return 0
    lax.fori_loop(0, BPW // L, gbody, 0)
    pltpu.sync_copy(out_v, out_hbm.at[pl.ds(base, BPW)])


@jax.jit
def _run(user_idx, item_idx, user_w, item_w):
    mesh = plsc.VectorSubcoreMesh(core_axis_name="c", subcore_axis_name="s")
    k1 = functools.partial(
        pl.kernel,
        out_type=(jax.ShapeDtypeStruct((B * D,), jnp.float32),
                  jax.ShapeDtypeStruct((B * D,), jnp.float32)),
        mesh=mesh,
        compiler_params=pltpu.CompilerParams(
            needs_layout_passes=False, use_tc_tiling_on_sc=True),
        scratch_types=[
            pltpu.VMEM((B,), jnp.int32),
            pltpu.VMEM((B,), jnp.int32),
            pltpu.VMEM((CAP,), jnp.int32),
            pltpu.VMEM((CAP,), jnp.int32),
            pltpu.VMEM((CAP,), jnp.int32),
            pltpu.VMEM((CAP,), jnp.int32),
            pltpu.VMEM((SCAP,), jnp.int32),
            pltpu.VMEM((SCAP,), jnp.int32),
            pltpu.VMEM((SCAP,), jnp.int32),
            pltpu.VMEM((SCAP,), jnp.int32),
            pltpu.SMEM((256,), jnp.int32),
            pltpu.SMEM((256,), jnp.int32),
            pltpu.SMEM((256,), jnp.int32),
            pltpu.SMEM((256,), jnp.int32),
            pltpu.VMEM((KQ, D, 128), jnp.float32),
            pltpu.VMEM((RS, D), jnp.float32),
            pltpu.SemaphoreType.DMA,
            pltpu.SemaphoreType.DMA,
        ],
    )(_gather_kernel)
    urows, vrows = k1(user_idx, item_idx, user_w.T, item_w.T)

    k2 = functools.partial(
        pl.kernel,
        out_type=jax.ShapeDtypeStruct((B,), jnp.float32),
        mesh=mesh,
        compiler_params=pltpu.CompilerParams(
            needs_layout_passes=False, use_tc_tiling_on_sc=True),
        scratch_types=[
            pltpu.VMEM((BPW * D,), jnp.float32),
            pltpu.VMEM((BPW * D,), jnp.float32),
            pltpu.VMEM((BPW,), jnp.float32),
            pltpu.SemaphoreType.DMA,
            pltpu.SemaphoreType.DMA,
        ],
    )(_dot_kernel)
    return k2(urows, vrows)


def kernel(user_idx, item_idx, user_w, item_w):
    return _run(user_idx, item_idx, user_w, item_w)
